# Initial kernel scaffold; baseline (speedup 1.0000x reference)
import jax
import jax.numpy as jnp
from jax.experimental import pallas as pl


def kernel(hours, hour_emb):
    raise NotImplementedError("write your pallas kernel here")


# speedup vs baseline: 1.6444x; 1.6444x over previous
import jax
import jax.numpy as jnp
from jax import lax
from jax.experimental import pallas as pl
from jax.experimental.pallas import tpu as pltpu
from jax.experimental.pallas import tpu_sc as plsc

D_MODEL = 128
GROUP = 128


def _make_lookup(B: int):
    info = plsc.get_sparse_core_info()
    NC, NS = info.num_cores, info.num_subcores
    NW = NC * NS
    assert B % (NW * GROUP) == 0
    rows_per_w = B // NW
    groups_per_w = rows_per_w // GROUP

    mesh = plsc.VectorSubcoreMesh(core_axis_name="c", subcore_axis_name="s")

    @jax.jit
    def lookup(table, idx):
        def body(table_hbm, idx_hbm, out_hbm, idx_v, rows_v, sem):
            wid = lax.axis_index("s") * NC + lax.axis_index("c")
            base = wid * rows_per_w

            def step(g, carry):
                off = base + g * GROUP
                pltpu.sync_copy(idx_hbm.at[pl.ds(off, GROUP)], idx_v)
                pltpu.async_copy(table_hbm.at[idx_v], rows_v, sem).wait()
                pltpu.sync_copy(rows_v, out_hbm.at[pl.ds(off, GROUP)])
                return carry

            lax.fori_loop(0, groups_per_w, step, 0, unroll=False)

        return pl.kernel(
            body,
            out_type=jax.ShapeDtypeStruct((B, D_MODEL), jnp.float32),
            mesh=mesh,
            scratch_types=[
                pltpu.VMEM((GROUP,), jnp.int32),
                pltpu.VMEM((GROUP, D_MODEL), jnp.float32),
                pltpu.SemaphoreType.DMA,
            ],
        )(table, idx)

    return lookup


def kernel(hours, hour_emb):
    B = hours.size
    flat = hours.reshape(B).astype(jnp.int32)
    out = _make_lookup(B)(hour_emb, flat)
    return out.reshape(*hours.shape, D_MODEL)


# speedup vs baseline: 1.6545x; 1.0061x over previous
import jax
import jax.numpy as jnp
from jax import lax
from jax.experimental import pallas as pl
from jax.experimental.pallas import tpu as pltpu
from jax.experimental.pallas import tpu_sc as plsc

D_MODEL = 128
GROUP = 128
CHUNK = 256
SUPER = 2048


def _make_lookup(B: int):
    info = plsc.get_sparse_core_info()
    NC, NS = info.num_cores, info.num_subcores
    NW = NC * NS
    rows_per_w = B // NW
    assert B == NW * rows_per_w and rows_per_w % SUPER == 0
    n_pairs = rows_per_w // (2 * CHUNK)
    pairs_per_super = SUPER // (2 * CHUNK)

    mesh = plsc.VectorSubcoreMesh(core_axis_name="c", subcore_axis_name="s")

    @jax.jit
    def lookup(table, idx):
        def body(table_hbm, idx_hbm, out_hbm, idx_s, rows0, rows1,
                 gsem0, gsem1, ssem0, ssem1):
            wid = lax.axis_index("s") * NC + lax.axis_index("c")
            base = wid * rows_per_w
            rows = (rows0, rows1)
            gsems = (gsem0, gsem1)
            ssems = (ssem0, ssem1)

            def pair(p, carry):
                @pl.when(p % pairs_per_super == 0)
                def _():
                    pltpu.sync_copy(
                        idx_hbm.at[pl.ds(base + (p // pairs_per_super) * SUPER,
                                         SUPER)],
                        idx_s)

                for b in (0, 1):
                    c = 2 * p + b
                    off = base + c * CHUNK
                    ioff = (c % (SUPER // CHUNK)) * CHUNK

                    @pl.when(p >= 1)
                    def _():
                        pltpu.make_async_copy(
                            rows[b], out_hbm.at[pl.ds(base, CHUNK)],
                            ssems[b]).wait()

                    g0 = pltpu.async_copy(
                        table_hbm.at[idx_s.at[pl.ds(ioff, GROUP)]],
                        rows[b].at[pl.ds(0, GROUP)], gsems[b])
                    g1 = pltpu.async_copy(
                        table_hbm.at[idx_s.at[pl.ds(ioff + GROUP, GROUP)]],
                        rows[b].at[pl.ds(GROUP, GROUP)], gsems[b])
                    g0.wait()
                    g1.wait()
                    pltpu.async_copy(rows[b], out_hbm.at[pl.ds(off, CHUNK)],
                                     ssems[b])
                return carry

            lax.fori_loop(0, n_pairs, pair, 0, unroll=False)
            for b in (0, 1):
                pltpu.make_async_copy(
                    rows[b], out_hbm.at[pl.ds(base, CHUNK)], ssems[b]).wait()

        return pl.kernel(
            body,
            out_type=jax.ShapeDtypeStruct((B, D_MODEL), jnp.float32),
            mesh=mesh,
            scratch_types=[
                pltpu.VMEM((SUPER,), jnp.int32),
                pltpu.VMEM((CHUNK, D_MODEL), jnp.float32),
                pltpu.VMEM((CHUNK, D_MODEL), jnp.float32),
                pltpu.SemaphoreType.DMA,
                pltpu.SemaphoreType.DMA,
                pltpu.SemaphoreType.DMA,
                pltpu.SemaphoreType.DMA,
            ],
        )(table, idx)

    return lookup


def kernel(hours, hour_emb):
    B = hours.size
    flat = hours.reshape(B).astype(jnp.int32)
    out = _make_lookup(B)(hour_emb, flat)
    return out.reshape(*hours.shape, D_MODEL)


# speedup vs baseline: 4.3767x; 2.6453x over previous
import jax
import jax.numpy as jnp
from jax import lax
from jax.experimental import pallas as pl
from jax.experimental.pallas import tpu as pltpu
from jax.experimental.pallas import tpu_sc as plsc

D_MODEL = 128
LANES = 16
CHUNK = 256
SUPER = 2048


def _make_lookup(B: int):
    info = plsc.get_sparse_core_info()
    NC, NS = info.num_cores, info.num_subcores
    NW = NC * NS
    rows_per_w = B // NW
    assert B == NW * rows_per_w and rows_per_w % SUPER == 0
    n_pairs = rows_per_w // (2 * CHUNK)
    pairs_per_super = SUPER // (2 * CHUNK)
    chunks_per_super = SUPER // CHUNK

    mesh = plsc.VectorSubcoreMesh(core_axis_name="c", subcore_axis_name="s")

    @jax.jit
    def lookup(table_flat, idx):
        def body(table_hbm, idx_hbm, out_hbm, table_v, idx_s, rows0, rows1,
                 ssem0, ssem1):
            wid = lax.axis_index("s") * NC + lax.axis_index("c")
            base = wid * rows_per_w
            rows = (rows0, rows1)
            ssems = (ssem0, ssem1)

            pltpu.sync_copy(table_hbm, table_v)

            def pair(p, carry):
                @pl.when(p % pairs_per_super == 0)
                def _():
                    pltpu.sync_copy(
                        idx_hbm.at[pl.ds(base + (p // pairs_per_super) * SUPER,
                                         SUPER)],
                        idx_s)

                for b in (0, 1):
                    c = 2 * p + b
                    off = base + c * CHUNK
                    ioff = (c % chunks_per_super) * CHUNK

                    @pl.when(p >= 1)
                    def _():
                        pltpu.make_async_copy(
                            rows[b], out_hbm.at[pl.ds(0, CHUNK * D_MODEL)],
                            ssems[b]).wait()

                    def group(g, carry2):
                        ivec = idx_s[pl.ds(ioff + g * LANES, LANES)]
                        for l in range(LANES):
                            src = ivec[l] * D_MODEL
                            dst = (g * LANES + l) * D_MODEL
                            for k in range(D_MODEL // LANES):
                                rows[b][pl.ds(dst + k * LANES, LANES)] = (
                                    table_v[pl.ds(src + k * LANES, LANES)])
                        return carry2

                    lax.fori_loop(0, CHUNK // LANES, group, 0, unroll=False)
                    pltpu.async_copy(
                        rows[b], out_hbm.at[pl.ds(off * D_MODEL,
                                                  CHUNK * D_MODEL)],
                        ssems[b])
                return carry

            lax.fori_loop(0, n_pairs, pair, 0, unroll=False)
            for b in (0, 1):
                pltpu.make_async_copy(
                    rows[b], out_hbm.at[pl.ds(0, CHUNK * D_MODEL)],
                    ssems[b]).wait()

        return pl.kernel(
            body,
            out_type=jax.ShapeDtypeStruct((B * D_MODEL,), jnp.float32),
            mesh=mesh,
            scratch_types=[
                pltpu.VMEM((24 * D_MODEL,), jnp.float32),
                pltpu.VMEM((SUPER,), jnp.int32),
                pltpu.VMEM((CHUNK * D_MODEL,), jnp.float32),
                pltpu.VMEM((CHUNK * D_MODEL,), jnp.float32),
                pltpu.SemaphoreType.DMA,
                pltpu.SemaphoreType.DMA,
            ],
        )(table_flat, idx)

    return lookup


def kernel(hours, hour_emb):
    B = hours.size
    flat = hours.reshape(B).astype(jnp.int32)
    out = _make_lookup(B)(hour_emb.reshape(-1), flat)
    return out.reshape(*hours.shape, D_MODEL)


# speedup vs baseline: 18.6078x; 4.2516x over previous
import jax
import jax.numpy as jnp
from jax import lax
from jax.experimental import pallas as pl
from jax.experimental.pallas import tpu as pltpu
from jax.experimental.pallas import tpu_sc as plsc

D_MODEL = 128
NUM_ROWS = 24
GROUP = 128
CHUNK = 256
SUPER = 2048


def _make_lookup(B: int):
    info = plsc.get_sparse_core_info()
    NC, NS = info.num_cores, info.num_subcores
    NW = NC * NS
    rows_per_w = B // NW
    assert B == NW * rows_per_w and rows_per_w % SUPER == 0
    n_pairs = rows_per_w // (2 * CHUNK)
    pairs_per_super = SUPER // (2 * CHUNK)
    chunks_per_super = SUPER // CHUNK

    mesh = plsc.VectorSubcoreMesh(core_axis_name="c", subcore_axis_name="s")

    @jax.jit
    def lookup(table, idx):
        def body(table_hbm, idx_hbm, out_hbm, table_sh, idx_s, rows0, rows1,
                 gsem0, gsem1, ssem0, ssem1):
            sid = lax.axis_index("s")
            wid = sid * NC + lax.axis_index("c")
            base = wid * rows_per_w
            rows = (rows0, rows1)
            gsems = (gsem0, gsem1)
            ssems = (ssem0, ssem1)

            @pl.when(sid == 0)
            def _():
                pltpu.sync_copy(table_hbm, table_sh)

            plsc.subcore_barrier()

            def pair(p, carry):
                @pl.when(p % pairs_per_super == 0)
                def _():
                    pltpu.sync_copy(
                        idx_hbm.at[pl.ds(base + (p // pairs_per_super) * SUPER,
                                         SUPER)],
                        idx_s)

                for b in (0, 1):
                    c = 2 * p + b
                    off = base + c * CHUNK
                    ioff = (c % chunks_per_super) * CHUNK

                    @pl.when(p >= 1)
                    def _():
                        pltpu.make_async_copy(
                            rows[b], out_hbm.at[pl.ds(0, CHUNK)],
                            ssems[b]).wait()

                    g0 = pltpu.async_copy(
                        table_sh.at[idx_s.at[pl.ds(ioff, GROUP)]],
                        rows[b].at[pl.ds(0, GROUP)], gsems[b])
                    g1 = pltpu.async_copy(
                        table_sh.at[idx_s.at[pl.ds(ioff + GROUP, GROUP)]],
                        rows[b].at[pl.ds(GROUP, GROUP)], gsems[b])
                    g0.wait()
                    g1.wait()
                    pltpu.async_copy(rows[b], out_hbm.at[pl.ds(off, CHUNK)],
                                     ssems[b])
                return carry

            lax.fori_loop(0, n_pairs, pair, 0, unroll=False)
            for b in (0, 1):
                pltpu.make_async_copy(
                    rows[b], out_hbm.at[pl.ds(0, CHUNK)], ssems[b]).wait()

        return pl.kernel(
            body,
            out_type=jax.ShapeDtypeStruct((B, D_MODEL), jnp.float32),
            mesh=mesh,
            scratch_types=[
                pltpu.VMEM_SHARED((NUM_ROWS, D_MODEL), jnp.float32),
                pltpu.VMEM((SUPER,), jnp.int32),
                pltpu.VMEM((CHUNK, D_MODEL), jnp.float32),
                pltpu.VMEM((CHUNK, D_MODEL), jnp.float32),
                pltpu.SemaphoreType.DMA,
                pltpu.SemaphoreType.DMA,
                pltpu.SemaphoreType.DMA,
                pltpu.SemaphoreType.DMA,
            ],
        )(table, idx)

    return lookup


def kernel(hours, hour_emb):
    B = hours.size
    flat = hours.reshape(B).astype(jnp.int32)
    out = _make_lookup(B)(hour_emb, flat)
    return out.reshape(*hours.shape, D_MODEL)
